# Initial kernel scaffold; baseline (speedup 1.0000x reference)
import jax
import jax.numpy as jnp
from jax.experimental import pallas as pl


def kernel(x, sigma, alpha_coeffs, mu_coeffs, phi_plus_coeffs, phi_minus_coeffs):
    raise NotImplementedError("write your pallas kernel here")


# speedup vs baseline: 3127.4876x; 3127.4876x over previous
import functools

import numpy as np
import jax
import jax.numpy as jnp
from jax import lax
from jax.experimental import pallas as pl
from jax.experimental.pallas import tpu as pltpu
from jax.experimental.pallas import tpu_sc as plsc

_NA = 80
_KS = 11
_KP = 101
_GD_S = np.float32(30.0 / (_KS - 1))
_GD_P = np.float32(10.0 / (_KP - 1))
_X_MIN = np.float32(-5.0)
_T_HI_P = np.float32(
    (np.float32(5.0 - 10.0 / (_KP - 1)) - _X_MIN) / _GD_P)
_T_HI_S = np.float32(np.float32(30.0 - 30.0 / (_KS - 1)) / _GD_S)

_NCH = 320
_NPIX = 128 * 128
_NW = 32
_CPW = _NCH // _NW


def _tables_body(sig_ref, ac_ref, mc_ref, pp_ref, pm_ref, a_ref, d_ref):
    f32 = jnp.float32
    sig = sig_ref[...]
    ks = lax.broadcasted_iota(jnp.int32, (_NCH, 128), 1).astype(f32)

    t = sig / _GD_S
    t_cl = jnp.clip(t, 0.0, _T_HI_S)
    idxf = jnp.floor(t_cl)
    fr = t - idxf
    w = jnp.where(ks == idxf, 1.0 - fr, 0.0) + jnp.where(ks == idxf + 1.0, fr, 0.0)
    alpha_s = jnp.sum(w * ac_ref[...], axis=1, keepdims=True)
    mu = jnp.sum(w * mc_ref[...], axis=1, keepdims=True)
    ea = jnp.exp(alpha_s)
    se = sig + np.float32(1e-5)
    s2 = (se * se) / (ea * ea)
    s1 = mu * s2

    ri = lax.broadcasted_iota(jnp.int32, (128, 128), 0)
    ci = lax.broadcasted_iota(jnp.int32, (128, 128), 1)
    cumsum_m = (ri < ci).astype(f32)
    rev_m = (ri + ci == _KP - 1).astype(f32)

    def project(cs):
        cn = pltpu.roll(cs, 127, 1)
        sl = jnp.clip((cn - cs) / _GD_P, 0.0, 1.0)
        sl = jnp.where((ks >= 1.0) & (ks <= np.float32(_KP - 3)), sl, 0.0)
        cum = lax.dot_general(sl, cumsum_m, (((1,), (0,)), ((), ())),
                              precision=lax.Precision.HIGHEST,
                              preferred_element_type=f32) * _GD_P
        rev = lax.dot_general(cum, rev_m, (((1,), (0,)), ((), ())),
                              precision=lax.Precision.HIGHEST,
                              preferred_element_type=f32)
        return 0.5 * (cum - rev)

    app = project(pp_ref[...])
    apm = project(pm_ref[...])
    dpp = pltpu.roll(app, 127, 1) - app
    dpm = pltpu.roll(apm, 127, 1) - apm

    a_ref[...] = s1 * app - s2 * apm
    d_ref[...] = s1 * dpp - s2 * dpm


def _sc_body(x_hbm, a_hbm, d_hbm, out_hbm, tbl_a, tbl_d, xbuf, obuf,
             sem_in, sem_out):
    cid = lax.axis_index("c")
    sid = lax.axis_index("s")
    wid = sid * 2 + cid
    ch0 = wid * _CPW
    e0 = ch0 * _NPIX

    pltpu.sync_copy(a_hbm.at[pl.ds(ch0 * 128, _CPW * 128)], tbl_a)
    pltpu.sync_copy(d_hbm.at[pl.ds(ch0 * 128, _CPW * 128)], tbl_d)

    in_copies = {0: pltpu.async_copy(x_hbm.at[pl.ds(e0, _NPIX)],
                                     xbuf.at[0], sem_in)}
    out_copies = {}
    for ch in range(_CPW):
        b = ch % 2
        if ch + 1 < _CPW:
            in_copies[ch + 1] = pltpu.async_copy(
                x_hbm.at[pl.ds(e0 + (ch + 1) * _NPIX, _NPIX)],
                xbuf.at[1 - b], sem_in)
        in_copies[ch].wait()
        if ch >= 2:
            out_copies[ch - 2].wait()

        @plsc.parallel_loop(0, _NPIX, step=16, unroll=8)
        def _inner(base, _b=b, _ch=ch):
            xv = xbuf[_b, pl.ds(base, 16)]
            t = (xv - _X_MIN) / _GD_P
            t_cl = jnp.clip(t, 0.0, _T_HI_P)
            idx = t_cl.astype(jnp.int32) + (_ch * 128)
            fr = t - t_cl.astype(jnp.int32).astype(jnp.float32)
            av = plsc.load_gather(tbl_a, [idx])
            dv = plsc.load_gather(tbl_d, [idx])
            obuf[_b, pl.ds(base, 16)] = av + fr * dv

        out_copies[ch] = pltpu.async_copy(
            obuf.at[b], out_hbm.at[pl.ds(e0 + ch * _NPIX, _NPIX)], sem_out)
    out_copies[_CPW - 2].wait()
    out_copies[_CPW - 1].wait()


def _build_tables(sigma, alpha_coeffs, mu_coeffs, phi_plus, phi_minus):
    f32 = jnp.float32
    sigb = jnp.broadcast_to(sigma.reshape(-1)[:, None], (_NCH, 128))
    ac = jnp.pad(alpha_coeffs, ((0, 0), (0, 128 - _KS)))
    acb = jnp.broadcast_to(ac[None], (_NCH // _NA, _NA, 128)).reshape(_NCH, 128)
    mcb = jnp.broadcast_to(jnp.pad(mu_coeffs, ((0, 0), (0, 128 - _KS))),
                           (_NCH, 128))
    ppb = jnp.broadcast_to(jnp.pad(phi_plus, ((0, 0), (0, 128 - _KP))),
                           (_NCH, 128))
    pmb = jnp.broadcast_to(jnp.pad(phi_minus, ((0, 0), (0, 128 - _KP))),
                           (_NCH, 128))
    return pl.pallas_call(
        _tables_body,
        out_shape=[jax.ShapeDtypeStruct((_NCH, 128), f32)] * 2,
    )(sigb, acb, mcb, ppb, pmb)


@functools.cache
def _sc_call():
    return pl.kernel(
        _sc_body,
        out_type=jax.ShapeDtypeStruct((_NCH * _NPIX,), jnp.float32),
        mesh=plsc.VectorSubcoreMesh(core_axis_name="c", subcore_axis_name="s",
                                    num_cores=2, num_subcores=16),
        scratch_types=[
            pltpu.VMEM((_CPW * 128,), jnp.float32),
            pltpu.VMEM((_CPW * 128,), jnp.float32),
            pltpu.VMEM((2, _NPIX), jnp.float32),
            pltpu.VMEM((2, _NPIX), jnp.float32),
            pltpu.SemaphoreType.DMA,
            pltpu.SemaphoreType.DMA,
        ],
        compiler_params=pltpu.CompilerParams(needs_layout_passes=False),
    )


def kernel(x, sigma, alpha_coeffs, mu_coeffs, phi_plus_coeffs, phi_minus_coeffs):
    a_tbl, d_tbl = _build_tables(sigma, alpha_coeffs, mu_coeffs,
                                 phi_plus_coeffs, phi_minus_coeffs)
    out = _sc_call()(x.reshape(-1), a_tbl.reshape(-1), d_tbl.reshape(-1))
    return out.reshape(x.shape)


# speedup vs baseline: 3323.3762x; 1.0626x over previous
import functools

import numpy as np
import jax
import jax.numpy as jnp
from jax import lax
from jax.experimental import pallas as pl
from jax.experimental.pallas import tpu as pltpu
from jax.experimental.pallas import tpu_sc as plsc

_NA = 80
_KS = 11
_KP = 101
_GD_S = np.float32(30.0 / (_KS - 1))
_GD_P = np.float32(10.0 / (_KP - 1))
_X_MIN = np.float32(-5.0)
_T_HI_P = np.float32(
    (np.float32(5.0 - 10.0 / (_KP - 1)) - _X_MIN) / _GD_P)
_T_HI_S = np.float32(np.float32(30.0 - 30.0 / (_KS - 1)) / _GD_S)

_NCH = 320
_NPIX = 128 * 128
_NW = 32
_CPW = _NCH // _NW


def _tables_body(sig_ref, ac_ref, mc_ref, pp_ref, pm_ref, a_ref, d_ref):
    f32 = jnp.float32
    sig = sig_ref[...]
    ks = lax.broadcasted_iota(jnp.int32, (_NCH, 128), 1).astype(f32)

    t = sig / _GD_S
    t_cl = jnp.clip(t, 0.0, _T_HI_S)
    idxf = jnp.floor(t_cl)
    fr = t - idxf
    w = jnp.where(ks == idxf, 1.0 - fr, 0.0) + jnp.where(ks == idxf + 1.0, fr, 0.0)
    alpha_s = jnp.sum(w * ac_ref[...], axis=1, keepdims=True)
    mu = jnp.sum(w * mc_ref[...], axis=1, keepdims=True)
    ea = jnp.exp(alpha_s)
    se = sig + np.float32(1e-5)
    s2 = (se * se) / (ea * ea)
    s1 = mu * s2

    ri = lax.broadcasted_iota(jnp.int32, (128, 128), 0)
    ci = lax.broadcasted_iota(jnp.int32, (128, 128), 1)
    cumsum_m = (ri < ci).astype(f32)
    rev_m = (ri + ci == _KP - 1).astype(f32)

    def project(cs):
        cn = pltpu.roll(cs, 127, 1)
        sl = jnp.clip((cn - cs) / _GD_P, 0.0, 1.0)
        sl = jnp.where((ks >= 1.0) & (ks <= np.float32(_KP - 3)), sl, 0.0)
        cum = lax.dot_general(sl, cumsum_m, (((1,), (0,)), ((), ())),
                              precision=lax.Precision.HIGHEST,
                              preferred_element_type=f32) * _GD_P
        rev = lax.dot_general(cum, rev_m, (((1,), (0,)), ((), ())),
                              precision=lax.Precision.HIGHEST,
                              preferred_element_type=f32)
        return 0.5 * (cum - rev)

    app = project(pp_ref[...])
    apm = project(pm_ref[...])
    dpp = pltpu.roll(app, 127, 1) - app
    dpm = pltpu.roll(apm, 127, 1) - apm

    rm = lax.broadcasted_iota(jnp.int32, (_NCH, 128), 0)
    kb = ks + (rm % _CPW).astype(f32) * np.float32(128.0)
    a = s1 * app - s2 * apm
    d = s1 * dpp - s2 * dpm
    a_ref[...] = a - kb * d
    d_ref[...] = d


def _sc_body(x_hbm, a_hbm, d_hbm, out_hbm, tbl_a, tbl_d, xbuf, obuf,
             sem_in, sem_out):
    cid = lax.axis_index("c")
    sid = lax.axis_index("s")
    wid = sid * 2 + cid
    ch0 = wid * _CPW
    e0 = ch0 * _NPIX

    pltpu.sync_copy(a_hbm.at[pl.ds(ch0 * 128, _CPW * 128)], tbl_a)
    pltpu.sync_copy(d_hbm.at[pl.ds(ch0 * 128, _CPW * 128)], tbl_d)

    in_copies = {0: pltpu.async_copy(x_hbm.at[pl.ds(e0, _NPIX)],
                                     xbuf.at[0], sem_in)}
    out_copies = {}
    for ch in range(_CPW):
        b = ch % 2
        if ch + 1 < _CPW:
            in_copies[ch + 1] = pltpu.async_copy(
                x_hbm.at[pl.ds(e0 + (ch + 1) * _NPIX, _NPIX)],
                xbuf.at[1 - b], sem_in)
        in_copies[ch].wait()
        if ch >= 2:
            out_copies[ch - 2].wait()

        c0 = np.float32(50.0 + ch * 128)
        lo = np.float32(ch * 128)
        hi = np.float32(ch * 128) + _T_HI_P

        @plsc.parallel_loop(0, _NPIX, step=16, unroll=8)
        def _inner(base, _b=b, _c0=c0, _lo=lo, _hi=hi):
            xv = xbuf[_b, pl.ds(base, 16)]
            t = xv * np.float32(10.0) + _c0
            idx = jnp.clip(t, _lo, _hi).astype(jnp.int32)
            av = plsc.load_gather(tbl_a, [idx])
            dv = plsc.load_gather(tbl_d, [idx])
            obuf[_b, pl.ds(base, 16)] = av + t * dv

        out_copies[ch] = pltpu.async_copy(
            obuf.at[b], out_hbm.at[pl.ds(e0 + ch * _NPIX, _NPIX)], sem_out)
    out_copies[_CPW - 2].wait()
    out_copies[_CPW - 1].wait()


def _build_tables(sigma, alpha_coeffs, mu_coeffs, phi_plus, phi_minus):
    f32 = jnp.float32
    sigb = jnp.broadcast_to(sigma.reshape(-1)[:, None], (_NCH, 128))
    ac = jnp.pad(alpha_coeffs, ((0, 0), (0, 128 - _KS)))
    acb = jnp.broadcast_to(ac[None], (_NCH // _NA, _NA, 128)).reshape(_NCH, 128)
    mcb = jnp.broadcast_to(jnp.pad(mu_coeffs, ((0, 0), (0, 128 - _KS))),
                           (_NCH, 128))
    ppb = jnp.broadcast_to(jnp.pad(phi_plus, ((0, 0), (0, 128 - _KP))),
                           (_NCH, 128))
    pmb = jnp.broadcast_to(jnp.pad(phi_minus, ((0, 0), (0, 128 - _KP))),
                           (_NCH, 128))
    return pl.pallas_call(
        _tables_body,
        out_shape=[jax.ShapeDtypeStruct((_NCH, 128), f32)] * 2,
    )(sigb, acb, mcb, ppb, pmb)


@functools.cache
def _sc_call():
    return pl.kernel(
        _sc_body,
        out_type=jax.ShapeDtypeStruct((_NCH * _NPIX,), jnp.float32),
        mesh=plsc.VectorSubcoreMesh(core_axis_name="c", subcore_axis_name="s",
                                    num_cores=2, num_subcores=16),
        scratch_types=[
            pltpu.VMEM((_CPW * 128,), jnp.float32),
            pltpu.VMEM((_CPW * 128,), jnp.float32),
            pltpu.VMEM((2, _NPIX), jnp.float32),
            pltpu.VMEM((2, _NPIX), jnp.float32),
            pltpu.SemaphoreType.DMA,
            pltpu.SemaphoreType.DMA,
        ],
        compiler_params=pltpu.CompilerParams(needs_layout_passes=False),
    )


def kernel(x, sigma, alpha_coeffs, mu_coeffs, phi_plus_coeffs, phi_minus_coeffs):
    a_tbl, d_tbl = _build_tables(sigma, alpha_coeffs, mu_coeffs,
                                 phi_plus_coeffs, phi_minus_coeffs)
    out = _sc_call()(x.reshape(-1), a_tbl.reshape(-1), d_tbl.reshape(-1))
    return out.reshape(x.shape)
